# Initial kernel scaffold; baseline (speedup 1.0000x reference)
import jax
import jax.numpy as jnp
from jax.experimental import pallas as pl


def kernel(x_s, x_t, edge_index_s, edge_index_t, xs_batch, xt_batch, Wx1, bx1, Wx2, bx2, Wx3, bx3, Wy1, by1, Wy2, by2, Wy3, by3, W_l1, b_l1, bn_g, bn_b, W_l2, b_l2):
    raise NotImplementedError("write your pallas kernel here")


# speedup vs baseline: 4.8642x; 4.8642x over previous
import functools

import jax
import jax.numpy as jnp
from jax import lax
from jax.experimental import pallas as pl
from jax.experimental.pallas import tpu as pltpu
from jax.experimental.pallas import tpu_sc as plsc

_N = 10000
_E = 320000
_G = 8
_NPAD = 10240
_R = 512
_NB = _NPAD // _R
_NSC = 2
_NT = 16
_C = 128
_W = 128
_NCHA = 79
_EA = _NSC * _NT * _NCHA * _C
_NCHB = 160
_IB = 32
_EB = _NT * _NCHB * _C
_RT = _NPAD // _NT

_f32 = jnp.float32
_PREC = lax.Precision.DEFAULT


def _dot(a, b):
    return lax.dot_general(a, b, (((1,), (0,)), ((), ())),
                           precision=_PREC, preferred_element_type=_f32)


def _mesh():
    return plsc.VectorSubcoreMesh(core_axis_name="c", subcore_axis_name="s",
                                  num_cores=_NSC, num_subcores=_NT)


def _zero_rows(rows_v):
    z = jnp.zeros((16,), _f32)

    def zrow(r, _):
        for k in range(_W // 16):
            rows_v[r, pl.ds(k * 16, 16)] = z
        return 0

    lax.fori_loop(0, _C, zrow, 0)


def _zero_acc(rows_v, acc_sh, s):
    def zacc(j, _):
        pltpu.sync_copy(rows_v, acc_sh.at[pl.ds(s * _RT + j * _C, _C)])
        return 0

    lax.fori_loop(0, _RT // _C, zacc, 0)


def _writeback(acc_sh, rows_v, out_hbm_c, s):
    def wb(j, _):
        pltpu.sync_copy(acc_sh.at[pl.ds(s * _RT + j * _C, _C)], rows_v)
        pltpu.sync_copy(rows_v, out_hbm_c.at[pl.ds(s * _RT + j * _C, _C)])
        return 0

    lax.fori_loop(0, _RT // _C, wb, 0)


@functools.partial(
    pl.kernel,
    out_type=jax.ShapeDtypeStruct((_NSC, _NPAD, _W), _f32),
    mesh=_mesh(),
    scratch_types=[
        pltpu.VMEM((_NCHA, _C), jnp.int32),
        pltpu.VMEM((_C, _W), _f32),
        pltpu.VMEM_SHARED((_NPAD, _W), _f32),
    ],
)
def _deg_kernel(src_hbm, out_hbm, src_v, val_v, acc_sh):
    c = lax.axis_index("c")
    s = lax.axis_index("s")
    w = s * _NSC + c
    pltpu.sync_copy(src_hbm.at[w], src_v)
    _zero_rows(val_v)
    _zero_acc(val_v, acc_sh, s)
    plsc.subcore_barrier()

    one = jnp.ones((16,), _f32)

    def orow(r, _):
        for k in range(_W // 16):
            val_v[r, pl.ds(k * 16, 16)] = one
        return 0

    lax.fori_loop(0, _C, orow, 0)

    def step(j, _):
        pltpu.sync_copy(val_v, acc_sh.at[src_v.at[j]], add=True)
        return 0

    lax.fori_loop(0, _NCHA, step, 0)
    plsc.subcore_barrier()
    _zero_rows(val_v)
    _writeback(acc_sh, val_v, out_hbm.at[c], s)


@functools.partial(
    pl.kernel,
    out_type=jax.ShapeDtypeStruct((_NSC, _NPAD, _W), _f32),
    mesh=_mesh(),
    scratch_types=[
        pltpu.VMEM((_NCHA, _C), jnp.int32),
        pltpu.VMEM((_NCHA, _C), jnp.int32),
        pltpu.VMEM((_C, _W), _f32),
        pltpu.VMEM_SHARED((_NPAD, _W), _f32),
    ],
)
def _prop_edge_split(g_hbm, src_hbm, dst_hbm, out_hbm,
                     src_v, dst_v, rows_v, acc_sh):
    c = lax.axis_index("c")
    s = lax.axis_index("s")
    w = s * _NSC + c
    pltpu.sync_copy(src_hbm.at[w], src_v)
    pltpu.sync_copy(dst_hbm.at[w], dst_v)
    _zero_rows(rows_v)
    _zero_acc(rows_v, acc_sh, s)
    plsc.subcore_barrier()

    def step(j, _):
        pltpu.sync_copy(g_hbm.at[src_v.at[j]], rows_v)
        pltpu.sync_copy(rows_v, acc_sh.at[dst_v.at[j]], add=True)
        return 0

    lax.fori_loop(0, _NCHA, step, 0)
    plsc.subcore_barrier()
    _writeback(acc_sh, rows_v, out_hbm.at[c], s)


@functools.partial(
    pl.kernel,
    out_type=jax.ShapeDtypeStruct((_NSC, _NPAD, _W), _f32),
    mesh=_mesh(),
    scratch_types=[
        pltpu.VMEM((_IB, _C), jnp.int32),
        pltpu.VMEM((_IB, _C), jnp.int32),
        pltpu.VMEM((_C, _W), _f32),
        pltpu.VMEM_SHARED((_NPAD, _W), _f32),
    ],
)
def _prop_feat_split(g_hbm, src_hbm, dst_hbm, out_hbm,
                     src_v, dst_v, rows_v, acc_sh):
    c = lax.axis_index("c")
    s = lax.axis_index("s")
    _zero_rows(rows_v)
    _zero_acc(rows_v, acc_sh, s)
    plsc.subcore_barrier()

    def blk(bi, _):
        pltpu.sync_copy(src_hbm.at[s].at[pl.ds(bi * _IB, _IB)], src_v)
        pltpu.sync_copy(dst_hbm.at[s].at[pl.ds(bi * _IB, _IB)], dst_v)

        def step(j, _):
            pltpu.sync_copy(g_hbm.at[c].at[src_v.at[j]], rows_v)
            pltpu.sync_copy(rows_v, acc_sh.at[dst_v.at[j]], add=True)
            return 0

        lax.fori_loop(0, _IB, step, 0)
        return 0

    lax.fori_loop(0, _NCHB // _IB, blk, 0)
    plsc.subcore_barrier()
    _writeback(acc_sh, rows_v, out_hbm.at[c], s)


def _pre_body(deg_ref, x_ref, dinv_ref, g_ref):
    i = pl.program_id(0)
    deg = deg_ref[0, :, 0:1] + deg_ref[1, :, 0:1]
    rows = i * _R + lax.broadcasted_iota(jnp.int32, (_R, 1), 0)
    valid = (rows < _N) & (deg > 0)
    dinv = jnp.where(valid, 1.0 / jnp.sqrt(jnp.maximum(deg, 1.0)), 0.0)
    dinv_ref[...] = dinv
    g_ref[...] = dinv * x_ref[...]


def _pre_call(degp, x):
    return pl.pallas_call(
        _pre_body,
        grid=(_NB,),
        in_specs=[
            pl.BlockSpec((_NSC, _R, _W), lambda i: (0, i, 0)),
            pl.BlockSpec((_R, 128), lambda i: (i, 0)),
        ],
        out_specs=[
            pl.BlockSpec((_R, 1), lambda i: (i, 0)),
            pl.BlockSpec((_R, 128), lambda i: (i, 0)),
        ],
        out_shape=[
            jax.ShapeDtypeStruct((_NPAD, 1), _f32),
            jax.ShapeDtypeStruct((_NPAD, 128), _f32),
        ],
    )(degp, x)


def _mid_a_body(h_ref, a_ref, dinv_ref, w0_ref, w1_ref, part_ref, g2_ref):
    dinv = dinv_ref[...]
    t1 = -dinv * (a_ref[0] + a_ref[1])
    part_ref[...] = _dot(h_ref[...], w0_ref[...]) + _dot(t1, w1_ref[...])
    g2_ref[...] = dinv * t1


def _mid_a_call(h, a1, dinv, w0, w1):
    din, dout = w0.shape
    return pl.pallas_call(
        _mid_a_body,
        grid=(_NB,),
        in_specs=[
            pl.BlockSpec((_R, din), lambda i: (i, 0)),
            pl.BlockSpec((_NSC, _R, _W), lambda i: (0, i, 0)),
            pl.BlockSpec((_R, 1), lambda i: (i, 0)),
            pl.BlockSpec((din, dout), lambda i: (0, 0)),
            pl.BlockSpec((din, dout), lambda i: (0, 0)),
        ],
        out_specs=[
            pl.BlockSpec((_R, dout), lambda i: (i, 0)),
            pl.BlockSpec((_R, din), lambda i: (i, 0)),
        ],
        out_shape=[
            jax.ShapeDtypeStruct((_NPAD, dout), _f32),
            jax.ShapeDtypeStruct((_NPAD, din), _f32),
        ],
    )(h, a1, dinv, w0, w1)


def _mid_b_body(h_ref, a_ref, dinv_ref, w0_ref, w1_ref, part_ref, g2_ref):
    dinv = dinv_ref[...]
    t10 = -dinv * a_ref[0]
    t11 = -dinv * a_ref[1]
    part_ref[...] = (_dot(h_ref[...], w0_ref[...])
                     + _dot(t10, w1_ref[0]) + _dot(t11, w1_ref[1]))
    g2_ref[0] = dinv * t10
    g2_ref[1] = dinv * t11


def _mid_b_call(h, a1, dinv, w0, w1r):
    din, dout = w0.shape
    return pl.pallas_call(
        _mid_b_body,
        grid=(_NB,),
        in_specs=[
            pl.BlockSpec((_R, din), lambda i: (i, 0)),
            pl.BlockSpec((_NSC, _R, _W), lambda i: (0, i, 0)),
            pl.BlockSpec((_R, 1), lambda i: (i, 0)),
            pl.BlockSpec((din, dout), lambda i: (0, 0)),
            pl.BlockSpec((_NSC, _W, dout), lambda i: (0, 0, 0)),
        ],
        out_specs=[
            pl.BlockSpec((_R, dout), lambda i: (i, 0)),
            pl.BlockSpec((_NSC, _R, _W), lambda i: (0, i, 0)),
        ],
        out_shape=[
            jax.ShapeDtypeStruct((_NPAD, dout), _f32),
            jax.ShapeDtypeStruct((_NSC, _NPAD, _W), _f32),
        ],
    )(h, a1, dinv, w0, w1r)


def _fin_body(h_ref, a_ref, part_ref, dinv_ref, w2_ref, b_ref,
              hn_ref, gn_ref, in_split, gn_split):
    dinv = dinv_ref[...]
    if in_split:
        tx20 = -2.0 * dinv * a_ref[0] - h_ref[:, :_W]
        tx21 = -2.0 * dinv * a_ref[1] - h_ref[:, _W:]
        macc = _dot(tx20, w2_ref[0]) + _dot(tx21, w2_ref[1])
    else:
        tx2 = -2.0 * dinv * (a_ref[0] + a_ref[1]) - h_ref[...]
        macc = _dot(tx2, w2_ref[...])
    hn = jnp.maximum(part_ref[...] + macc + b_ref[...], 0.0)
    hn_ref[...] = hn
    if gn_ref is not None:
        if gn_split:
            gn_ref[0] = dinv * hn[:, :_W]
            gn_ref[1] = dinv * hn[:, _W:]
        else:
            gn_ref[...] = dinv * hn


def _fin_call(h, a2, part, dinv, w2, b, in_split, gn_mode):
    din = h.shape[1]
    dout = part.shape[1]
    w2_spec = (pl.BlockSpec((_NSC, _W, dout), lambda i: (0, 0, 0)) if in_split
               else pl.BlockSpec((din, dout), lambda i: (0, 0)))
    out_specs = [pl.BlockSpec((_R, dout), lambda i: (i, 0))]
    out_shape = [jax.ShapeDtypeStruct((_NPAD, dout), _f32)]
    if gn_mode == "flat":
        out_specs.append(pl.BlockSpec((_R, dout), lambda i: (i, 0)))
        out_shape.append(jax.ShapeDtypeStruct((_NPAD, dout), _f32))
    elif gn_mode == "split":
        out_specs.append(pl.BlockSpec((_NSC, _R, _W), lambda i: (0, i, 0)))
        out_shape.append(jax.ShapeDtypeStruct((_NSC, _NPAD, _W), _f32))

    if gn_mode == "none":
        def body(h_ref, a_ref, part_ref, dinv_ref, w2_ref, b_ref, hn_ref):
            _fin_body(h_ref, a_ref, part_ref, dinv_ref, w2_ref, b_ref,
                      hn_ref, None, in_split, False)
    else:
        gsp = gn_mode == "split"

        def body(h_ref, a_ref, part_ref, dinv_ref, w2_ref, b_ref,
                 hn_ref, gn_ref):
            _fin_body(h_ref, a_ref, part_ref, dinv_ref, w2_ref, b_ref,
                      hn_ref, gn_ref, in_split, gsp)

    res = pl.pallas_call(
        body,
        grid=(_NB,),
        in_specs=[
            pl.BlockSpec((_R, din), lambda i: (i, 0)),
            pl.BlockSpec((_NSC, _R, _W), lambda i: (0, i, 0)),
            pl.BlockSpec((_R, dout), lambda i: (i, 0)),
            pl.BlockSpec((_R, 1), lambda i: (i, 0)),
            w2_spec,
            pl.BlockSpec((1, dout), lambda i: (0, 0)),
        ],
        out_specs=out_specs,
        out_shape=out_shape,
    )(h, a2, part, dinv, w2, b)
    return res if gn_mode != "none" else (res[0], None)


def _head_body(h_ref, ids_ref, wl1_ref, bl1_ref, bng_ref, bnb_ref,
               wl2_ref, bl2_ref, out_ref, sig_ref, ssum, scnt):
    i = pl.program_id(0)

    @pl.when(i == 0)
    def _():
        ssum[...] = jnp.zeros_like(ssum)
        scnt[...] = jnp.zeros_like(scnt)

    ids = ids_ref[...]
    h = h_ref[...]
    oh = (ids == lax.broadcasted_iota(jnp.int32, (1, _G), 1)).astype(_f32)
    psum = jnp.stack([jnp.sum(jnp.where(ids == g, h, 0.0), axis=0)
                      for g in range(_G)])
    ssum[...] += psum
    scnt[...] += jnp.broadcast_to(jnp.sum(oh, axis=0)[:, None], scnt.shape)

    @pl.when(i == _NB - 1)
    def _():
        cnt = scnt[:, 0:1]
        xs = ssum[...] / jnp.maximum(cnt, 1.0)
        u = jnp.maximum(_dot(xs, wl1_ref[...]) + bl1_ref[...], 0.0)
        m = jnp.mean(u, axis=0, keepdims=True)
        d = u - m
        v = jnp.mean(d * d, axis=0, keepdims=True)
        bn = d * lax.rsqrt(v + 1e-5) * bng_ref[...] + bnb_ref[...]
        o = _dot(bn, wl2_ref[...]) + bl2_ref[...]
        out_ref[...] = o
        sig_ref[...] = 1.0 / (1.0 + jnp.exp(-o))


def _head_call(h3, ids, wl1, bl1, bng, bnb, wl2p, bl2p):
    dpad = wl2p.shape[1]
    return pl.pallas_call(
        _head_body,
        grid=(_NB,),
        in_specs=[
            pl.BlockSpec((_R, 512), lambda i: (i, 0)),
            pl.BlockSpec((_R, 1), lambda i: (i, 0)),
            pl.BlockSpec((512, 1024), lambda i: (0, 0)),
            pl.BlockSpec((1, 1024), lambda i: (0, 0)),
            pl.BlockSpec((1, 1024), lambda i: (0, 0)),
            pl.BlockSpec((1, 1024), lambda i: (0, 0)),
            pl.BlockSpec((1024, dpad), lambda i: (0, 0)),
            pl.BlockSpec((1, dpad), lambda i: (0, 0)),
        ],
        out_specs=[
            pl.BlockSpec((_G, dpad), lambda i: (0, 0)),
            pl.BlockSpec((_G, dpad), lambda i: (0, 0)),
        ],
        out_shape=[
            jax.ShapeDtypeStruct((_G, dpad), _f32),
            jax.ShapeDtypeStruct((_G, dpad), _f32),
        ],
        scratch_shapes=[
            pltpu.VMEM((_G, 512), _f32),
            pltpu.VMEM((_G, 128), _f32),
        ],
    )(h3, ids, wl1, bl1, bng, bnb, wl2p, bl2p)


def kernel(x_s, x_t, edge_index_s, edge_index_t, xs_batch, xt_batch,
           Wx1, bx1, Wx2, bx2, Wx3, bx3, Wy1, by1, Wy2, by2, Wy3, by3,
           W_l1, b_l1, bn_g, bn_b, W_l2, b_l2):
    n = x_s.shape[0]
    x = jnp.pad(x_s, ((0, _NPAD - n), (0, 0)))
    ids = jnp.pad(xs_batch, (0, _NPAD - n),
                  constant_values=_G).reshape(_NPAD, 1)
    dst, src = lax.sort((edge_index_s[1], edge_index_s[0]),
                        num_keys=1, is_stable=True)
    pad_idx = _NPAD - 1
    srcA = jnp.pad(src, (0, _EA - _E),
                   constant_values=pad_idx).reshape(_NSC * _NT, _NCHA, _C)
    dstA = jnp.pad(dst, (0, _EA - _E),
                   constant_values=pad_idx).reshape(_NSC * _NT, _NCHA, _C)
    srcB = jnp.pad(src, (0, _EB - _E),
                   constant_values=pad_idx).reshape(_NT, _NCHB, _C)
    dstB = jnp.pad(dst, (0, _EB - _E),
                   constant_values=pad_idx).reshape(_NT, _NCHB, _C)

    degp = _deg_kernel(srcA)
    dinv, g = _pre_call(degp, x)

    a1 = _prop_edge_split(g, srcA, dstA)
    part, g2 = _mid_a_call(x, a1, dinv, Wx1[0], Wx1[1])
    a2 = _prop_edge_split(g2, srcA, dstA)
    h, g = _fin_call(x, a2, part, dinv, Wx1[2], bx1.reshape(1, -1),
                     in_split=False, gn_mode="flat")

    a1 = _prop_edge_split(g, srcA, dstA)
    part, g2 = _mid_a_call(h, a1, dinv, Wx2[0], Wx2[1])
    a2 = _prop_edge_split(g2, srcA, dstA)
    h, g = _fin_call(h, a2, part, dinv, Wx2[2], bx2.reshape(1, -1),
                     in_split=False, gn_mode="split")

    w31 = Wx3[1].reshape(_NSC, _W, 512)
    w32 = Wx3[2].reshape(_NSC, _W, 512)
    a1 = _prop_feat_split(g, srcB, dstB)
    part, g2 = _mid_b_call(h, a1, dinv, Wx3[0], w31)
    a2 = _prop_feat_split(g2, srcB, dstB)
    h, _ = _fin_call(h, a2, part, dinv, w32, bx3.reshape(1, -1),
                     in_split=True, gn_mode="none")

    dpad = 1408
    wl2p = jnp.pad(W_l2, ((0, 0), (0, dpad - W_l2.shape[1])))
    bl2p = jnp.pad(b_l2, (0, dpad - b_l2.shape[0])).reshape(1, dpad)
    out, sig = _head_call(h, ids, W_l1, b_l1.reshape(1, -1),
                          bn_g.reshape(1, -1), bn_b.reshape(1, -1),
                          wl2p, bl2p)
    return out[:, :1317], sig[:, :1317]
